# baseline (speedup 1.0000x reference)
import functools

import jax
import jax.numpy as jnp
from jax import lax
from jax.experimental import pallas as pl
from jax.experimental.pallas import tpu as pltpu
from jax.experimental.pallas import tpu_sc as plsc

BATCH = 4096
MAXLEN = 200
EMBED = 64
LANES = 16

_info = plsc.get_sparse_core_info()
NC, NS = _info.num_cores, _info.num_subcores
NW = NC * NS
BPW = BATCH // NW
VPR = EMBED // LANES
PAIR = 2 * EMBED


def _body(xh_hbm, xoff_hbm, pos_hbm, tok_hbm, out_hbm,
          idx_v, par_v, pos_v, rows0, rows1, obuf, g0, g1):
    w = lax.axis_index("s") * NC + lax.axis_index("c")
    b0 = w * BPW
    pltpu.sync_copy(xh_hbm.at[:, pl.ds(b0, BPW)], idx_v)
    pltpu.sync_copy(xoff_hbm.at[:, pl.ds(b0, BPW)], par_v)
    pltpu.sync_copy(pos_hbm, pos_v)
    rows = (rows0, rows1)
    sems = (g0, g1)

    def fetch_start(t, b):
        pltpu.make_async_copy(tok_hbm.at[idx_v.at[t]], rows[b], sems[b]).start()

    def fetch_wait(t, b):
        pltpu.make_async_copy(tok_hbm.at[idx_v.at[t]], rows[b], sems[b]).wait()

    def add_pos(t, b, side):
        rbuf = rows[b]
        pv = [pos_v[t, pl.ds(k * LANES, LANES)] for k in range(VPR)]

        def grp(g, carry):
            parv = par_v[t, pl.ds(g * LANES, LANES)]
            base = g * LANES
            for j in range(LANES):
                off = parv[j]
                r = base + j
                for k in range(VPR):
                    src = rbuf[r, pl.ds(off + k * LANES, LANES)]
                    obuf[r, pl.ds(side * EMBED + k * LANES, LANES)] = src + pv[k]
            return carry

        lax.fori_loop(0, BPW // LANES, grp, 0)

    def store(m):
        pltpu.sync_copy(obuf, out_hbm.at[pl.ds(b0, BPW), m])

    fetch_start(0, 0)

    def outer(m, carry):
        t0 = m * 2
        fetch_start(t0 + 1, 1)
        fetch_wait(t0, 0)
        add_pos(t0, 0, 0)

        @pl.when(t0 + 2 < MAXLEN)
        def _():
            fetch_start(t0 + 2, 0)

        fetch_wait(t0 + 1, 1)
        add_pos(t0 + 1, 1, 1)
        store(m)
        return carry

    lax.fori_loop(0, MAXLEN // 2, outer, 0)


@jax.jit
def kernel(x, token_table, pos_table):
    B, T = x.shape
    V, D = token_table.shape
    assert (B, T, D) == (BATCH, MAXLEN, EMBED)
    x32 = x.astype(jnp.int32)
    xh = lax.shift_right_logical(x32, 1).T
    xoff = lax.shift_left(jnp.bitwise_and(x32, 1), 6).T
    tok2 = token_table.reshape(V // 2, PAIR)
    pos2 = jnp.pad(pos_table, ((0, 0), (0, PAIR - D)))

    run = pl.kernel(
        _body,
        out_type=jax.ShapeDtypeStruct((B, T // 2, PAIR), jnp.float32),
        mesh=plsc.VectorSubcoreMesh(core_axis_name="c", subcore_axis_name="s"),
        compiler_params=pltpu.CompilerParams(use_tc_tiling_on_sc=True),
        scratch_types=[
            pltpu.VMEM((MAXLEN, BPW), jnp.int32),
            pltpu.VMEM((MAXLEN, BPW), jnp.int32),
            pltpu.VMEM((MAXLEN, PAIR), jnp.float32),
            pltpu.VMEM((BPW, PAIR), jnp.float32),
            pltpu.VMEM((BPW, PAIR), jnp.float32),
            pltpu.VMEM((BPW, PAIR), jnp.float32),
            pltpu.SemaphoreType.DMA,
            pltpu.SemaphoreType.DMA,
        ],
    )
    out = run(xh, xoff, pos2, tok2)
    return out.reshape(B, T, D)
